# Initial kernel scaffold; baseline (speedup 1.0000x reference)
import jax
import jax.numpy as jnp
from jax.experimental import pallas as pl


def kernel(x, rand_inds):
    raise NotImplementedError("write your pallas kernel here")


# speedup vs baseline: 8.5141x; 8.5141x over previous
import functools

import jax
import jax.numpy as jnp
from jax import lax
from jax.experimental import pallas as pl
from jax.experimental.pallas import tpu as pltpu
from jax.experimental.pallas import tpu_sc as plsc

N = 4096
Q = 4096
D = 256
OUT_ROWS = 8192

NC, NS, L = 2, 16, 16
NW = NC * NS
BPW = Q // NW

_sc_mesh = plsc.VectorSubcoreMesh(core_axis_name="c", subcore_axis_name="s")


@functools.partial(
    pl.kernel,
    mesh=_sc_mesh,
    out_type=jax.ShapeDtypeStruct((Q, D), jnp.float32),
    scratch_types=[
        pltpu.VMEM((BPW,), jnp.int32),
        pltpu.VMEM((BPW, D), jnp.float32),
        pltpu.SemaphoreType.DMA,
    ],
)
def _sc_gather_q(x_hbm, inds_hbm, q_hbm, idx_v, rows_v, sem):
    wid = lax.axis_index("s") * NC + lax.axis_index("c")
    base = wid * BPW
    pltpu.sync_copy(inds_hbm.at[pl.ds(base, BPW)], idx_v)
    pltpu.async_copy(x_hbm.at[idx_v], rows_v, sem).wait()
    pltpu.sync_copy(rows_v, q_hbm.at[pl.ds(base, BPW)])


BQ = 128
GRID = Q // BQ


def _tc_topk_body(q_ref, x_ref, qsq_ref, xsq_ref, i1_ref, i2_ref, i3_ref):
    q = q_ref[...].astype(jnp.bfloat16)
    x = x_ref[...].astype(jnp.bfloat16)
    mm = lax.dot_general(q, x, (((1,), (1,)), ((), ())),
                         preferred_element_type=jnp.float32)
    s = (qsq_ref[...] - 2.0 * mm) + xsq_ref[...]
    cols = lax.broadcasted_iota(jnp.int32, (BQ, N), 1)
    for t, ref in enumerate((i1_ref, i2_ref, i3_ref)):
        m = jnp.min(s, axis=1, keepdims=True)
        idx = jnp.min(jnp.where(s == m, cols, N), axis=1, keepdims=True)
        ref[...] = idx
        if t < 2:
            s = jnp.where(cols == idx, jnp.float32(jnp.inf), s)


_tc_topk = pl.pallas_call(
    _tc_topk_body,
    grid=(GRID,),
    in_specs=[
        pl.BlockSpec((BQ, D), lambda i: (i, 0)),
        pl.BlockSpec((N, D), lambda i: (0, 0)),
        pl.BlockSpec((BQ, 1), lambda i: (i, 0)),
        pl.BlockSpec((1, N), lambda i: (0, 0)),
    ],
    out_specs=[
        pl.BlockSpec((BQ, 1), lambda i: (i, 0)),
        pl.BlockSpec((BQ, 1), lambda i: (i, 0)),
        pl.BlockSpec((BQ, 1), lambda i: (i, 0)),
    ],
    out_shape=[jax.ShapeDtypeStruct((Q, 1), jnp.int32)] * 3,
)


@functools.partial(
    pl.kernel,
    mesh=_sc_mesh,
    out_type=jax.ShapeDtypeStruct((OUT_ROWS, D), jnp.float32),
    scratch_types=[
        pltpu.VMEM((BPW,), jnp.int32),
        pltpu.VMEM((BPW,), jnp.int32),
        pltpu.VMEM((BPW,), jnp.int32),
        pltpu.VMEM((BPW, D), jnp.float32),
        pltpu.VMEM((BPW, D), jnp.float32),
        pltpu.VMEM((BPW, D), jnp.float32),
        pltpu.SemaphoreType.DMA,
    ],
)
def _sc_mean(x_hbm, i1_hbm, i2_hbm, i3_hbm, out_hbm,
             i1_v, i2_v, i3_v, a_v, b_v, c_v, sem):
    wid = lax.axis_index("s") * NC + lax.axis_index("c")
    base = wid * BPW
    pltpu.sync_copy(x_hbm.at[pl.ds(base, BPW)], a_v)
    pltpu.sync_copy(a_v, out_hbm.at[pl.ds(base, BPW)])
    pltpu.sync_copy(i1_hbm.at[pl.ds(base, BPW)], i1_v)
    pltpu.sync_copy(i2_hbm.at[pl.ds(base, BPW)], i2_v)
    pltpu.sync_copy(i3_hbm.at[pl.ds(base, BPW)], i3_v)
    ca = pltpu.async_copy(x_hbm.at[i1_v], a_v, sem)
    cb = pltpu.async_copy(x_hbm.at[i2_v], b_v, sem)
    cc = pltpu.async_copy(x_hbm.at[i3_v], c_v, sem)
    ca.wait()
    cb.wait()
    cc.wait()
    third = jnp.float32(1.0 / 3.0)

    def row_body(r, carry):
        for ch in range(D // L):
            sl = pl.ds(ch * L, L)
            a_v[r, sl] = (a_v[r, sl] + b_v[r, sl] + c_v[r, sl]) * third
        return carry

    lax.fori_loop(0, BPW, row_body, 0)
    pltpu.sync_copy(a_v, out_hbm.at[pl.ds(N + base, BPW)])


def kernel(x, rand_inds):
    q = _sc_gather_q(x, rand_inds)
    x_sq = jnp.sum(x * x, axis=1).reshape(1, N)
    q_sq = jnp.sum(q * q, axis=1).reshape(Q, 1)
    i1, i2, i3 = _tc_topk(q, x, q_sq, x_sq)
    return _sc_mean(x, i1.reshape(Q), i2.reshape(Q), i3.reshape(Q))


# speedup vs baseline: 8.9946x; 1.0564x over previous
import functools

import jax
import jax.numpy as jnp
from jax import lax
from jax.experimental import pallas as pl
from jax.experimental.pallas import tpu as pltpu
from jax.experimental.pallas import tpu_sc as plsc

N = 4096
Q = 4096
D = 256
OUT_ROWS = 8192

NC, NS, L = 2, 16, 16
NW = NC * NS
BPW = Q // NW

_sc_mesh = plsc.VectorSubcoreMesh(core_axis_name="c", subcore_axis_name="s")


@functools.partial(
    pl.kernel,
    mesh=_sc_mesh,
    out_type=jax.ShapeDtypeStruct((Q, D), jnp.float32),
    scratch_types=[
        pltpu.VMEM((BPW,), jnp.int32),
        pltpu.VMEM((BPW, D), jnp.float32),
        pltpu.SemaphoreType.DMA,
    ],
)
def _sc_gather_q(x_hbm, inds_hbm, q_hbm, idx_v, rows_v, sem):
    wid = lax.axis_index("s") * NC + lax.axis_index("c")
    base = wid * BPW
    pltpu.sync_copy(inds_hbm.at[pl.ds(base, BPW)], idx_v)
    pltpu.async_copy(x_hbm.at[idx_v], rows_v, sem).wait()
    pltpu.sync_copy(rows_v, q_hbm.at[pl.ds(base, BPW)])


BQ = 128
GRID = Q // BQ


def _tc_topk_body(q_ref, x_ref, xsq_ref, i1_ref, i2_ref, i3_ref):
    qf = q_ref[...]
    q = qf.astype(jnp.bfloat16)
    x = x_ref[...].astype(jnp.bfloat16)
    mm = lax.dot_general(q, x, (((1,), (1,)), ((), ())),
                         preferred_element_type=jnp.float32)
    qsq = jnp.sum(qf * qf, axis=1, keepdims=True)
    s = (qsq - 2.0 * mm) + xsq_ref[...]
    cols = lax.broadcasted_iota(jnp.int32, (BQ, N), 1)
    for t, ref in enumerate((i1_ref, i2_ref, i3_ref)):
        m = jnp.min(s, axis=1, keepdims=True)
        idx = jnp.min(jnp.where(s == m, cols, N), axis=1, keepdims=True)
        ref[...] = idx[:, 0]
        if t < 2:
            s = jnp.where(cols == idx, jnp.float32(jnp.inf), s)


_tc_topk = pl.pallas_call(
    _tc_topk_body,
    grid=(GRID,),
    in_specs=[
        pl.BlockSpec((BQ, D), lambda i: (i, 0)),
        pl.BlockSpec((N, D), lambda i: (0, 0)),
        pl.BlockSpec((1, N), lambda i: (0, 0)),
    ],
    out_specs=[
        pl.BlockSpec((BQ,), lambda i: (i,)),
        pl.BlockSpec((BQ,), lambda i: (i,)),
        pl.BlockSpec((BQ,), lambda i: (i,)),
    ],
    out_shape=[jax.ShapeDtypeStruct((Q,), jnp.int32)] * 3,
)


@functools.partial(
    pl.kernel,
    mesh=_sc_mesh,
    out_type=jax.ShapeDtypeStruct((OUT_ROWS, D), jnp.float32),
    scratch_types=[
        pltpu.VMEM((BPW,), jnp.int32),
        pltpu.VMEM((BPW,), jnp.int32),
        pltpu.VMEM((BPW,), jnp.int32),
        pltpu.VMEM((BPW, D), jnp.float32),
        pltpu.VMEM((BPW, D), jnp.float32),
        pltpu.VMEM((BPW, D), jnp.float32),
        pltpu.SemaphoreType.DMA,
    ],
)
def _sc_mean(x_hbm, i1_hbm, i2_hbm, i3_hbm, out_hbm,
             i1_v, i2_v, i3_v, a_v, b_v, c_v, sem):
    wid = lax.axis_index("s") * NC + lax.axis_index("c")
    base = wid * BPW
    pltpu.sync_copy(x_hbm.at[pl.ds(base, BPW)], a_v)
    pltpu.sync_copy(a_v, out_hbm.at[pl.ds(base, BPW)])
    pltpu.sync_copy(i1_hbm.at[pl.ds(base, BPW)], i1_v)
    pltpu.sync_copy(i2_hbm.at[pl.ds(base, BPW)], i2_v)
    pltpu.sync_copy(i3_hbm.at[pl.ds(base, BPW)], i3_v)
    ca = pltpu.async_copy(x_hbm.at[i1_v], a_v, sem)
    cb = pltpu.async_copy(x_hbm.at[i2_v], b_v, sem)
    cc = pltpu.async_copy(x_hbm.at[i3_v], c_v, sem)
    ca.wait()
    cb.wait()
    cc.wait()
    third = jnp.float32(1.0 / 3.0)

    def row_body(r, carry):
        for ch in range(D // L):
            sl = pl.ds(ch * L, L)
            a_v[r, sl] = (a_v[r, sl] + b_v[r, sl] + c_v[r, sl]) * third
        return carry

    lax.fori_loop(0, BPW, row_body, 0)
    pltpu.sync_copy(a_v, out_hbm.at[pl.ds(N + base, BPW)])


def kernel(x, rand_inds):
    q = _sc_gather_q(x, rand_inds)
    x_sq = jnp.sum(x * x, axis=1).reshape(1, N)
    i1, i2, i3 = _tc_topk(q, x, x_sq)
    return _sc_mean(x, i1, i2, i3)


# speedup vs baseline: 11.0513x; 1.2287x over previous
import functools

import jax
import jax.numpy as jnp
from jax import lax
from jax.experimental import pallas as pl
from jax.experimental.pallas import tpu as pltpu
from jax.experimental.pallas import tpu_sc as plsc

N = 4096
Q = 4096
D = 256
OUT_ROWS = 8192

NC, NS, L = 2, 16, 16
NW = NC * NS
BPW = Q // NW

_sc_mesh = plsc.VectorSubcoreMesh(core_axis_name="c", subcore_axis_name="s")


@functools.partial(
    pl.kernel,
    mesh=_sc_mesh,
    out_type=jax.ShapeDtypeStruct((Q, D), jnp.float32),
    scratch_types=[
        pltpu.VMEM((BPW,), jnp.int32),
        pltpu.VMEM((BPW, D), jnp.float32),
        pltpu.SemaphoreType.DMA,
    ],
)
def _sc_gather_q(x_hbm, inds_hbm, q_hbm, idx_v, rows_v, sem):
    wid = lax.axis_index("s") * NC + lax.axis_index("c")
    base = wid * BPW
    pltpu.sync_copy(inds_hbm.at[pl.ds(base, BPW)], idx_v)
    pltpu.async_copy(x_hbm.at[idx_v], rows_v, sem).wait()
    pltpu.sync_copy(rows_v, q_hbm.at[pl.ds(base, BPW)])


BQ = 256
GRID = Q // BQ


def _tc_topk_body(q_ref, x_ref, xsq_ref, i1_ref, i2_ref, i3_ref):
    qf = q_ref[...]
    q = qf.astype(jnp.bfloat16)
    x = x_ref[...].astype(jnp.bfloat16)
    mm = lax.dot_general(q, x, (((1,), (1,)), ((), ())),
                         preferred_element_type=jnp.float32)
    qsq = jnp.sum(qf * qf, axis=1, keepdims=True)
    s = (qsq - 2.0 * mm) + xsq_ref[...]
    cols = lax.broadcasted_iota(jnp.int32, (BQ, N), 1)
    for t, ref in enumerate((i1_ref, i2_ref, i3_ref)):
        m = jnp.min(s, axis=1, keepdims=True)
        idx = jnp.min(jnp.where(s == m, cols, N), axis=1, keepdims=True)
        ref[...] = idx[:, 0]
        if t < 2:
            s = jnp.where(cols == idx, jnp.float32(jnp.inf), s)


_tc_topk = pl.pallas_call(
    _tc_topk_body,
    grid=(GRID,),
    in_specs=[
        pl.BlockSpec((BQ, D), lambda i: (i, 0)),
        pl.BlockSpec((N, D), lambda i: (0, 0)),
        pl.BlockSpec((1, N), lambda i: (0, 0)),
    ],
    out_specs=[
        pl.BlockSpec((BQ,), lambda i: (i,)),
        pl.BlockSpec((BQ,), lambda i: (i,)),
        pl.BlockSpec((BQ,), lambda i: (i,)),
    ],
    out_shape=[jax.ShapeDtypeStruct((Q,), jnp.int32)] * 3,
)


@functools.partial(
    pl.kernel,
    mesh=_sc_mesh,
    out_type=jax.ShapeDtypeStruct((OUT_ROWS, D), jnp.float32),
    scratch_types=[
        pltpu.VMEM((BPW,), jnp.int32),
        pltpu.VMEM((BPW,), jnp.int32),
        pltpu.VMEM((BPW,), jnp.int32),
        pltpu.VMEM((BPW, D), jnp.float32),
        pltpu.VMEM((BPW, D), jnp.float32),
        pltpu.VMEM((BPW, D), jnp.float32),
        pltpu.SemaphoreType.DMA,
    ],
)
def _sc_mean(x_hbm, i1_hbm, i2_hbm, i3_hbm, out_hbm,
             i1_v, i2_v, i3_v, a_v, b_v, c_v, sem):
    wid = lax.axis_index("s") * NC + lax.axis_index("c")
    base = wid * BPW
    pltpu.sync_copy(x_hbm.at[pl.ds(base, BPW)], a_v)
    pltpu.sync_copy(a_v, out_hbm.at[pl.ds(base, BPW)])
    pltpu.sync_copy(i1_hbm.at[pl.ds(base, BPW)], i1_v)
    pltpu.sync_copy(i2_hbm.at[pl.ds(base, BPW)], i2_v)
    pltpu.sync_copy(i3_hbm.at[pl.ds(base, BPW)], i3_v)
    ca = pltpu.async_copy(x_hbm.at[i1_v], a_v, sem)
    cb = pltpu.async_copy(x_hbm.at[i2_v], b_v, sem)
    cc = pltpu.async_copy(x_hbm.at[i3_v], c_v, sem)
    ca.wait()
    cb.wait()
    cc.wait()
    third = jnp.float32(1.0 / 3.0)

    def row_body(r, carry):
        for ch in range(D // L):
            sl = pl.ds(ch * L, L)
            a_v[r, sl] = (a_v[r, sl] + b_v[r, sl] + c_v[r, sl]) * third
        return carry

    lax.fori_loop(0, BPW, row_body, 0)
    pltpu.sync_copy(a_v, out_hbm.at[pl.ds(N + base, BPW)])


def kernel(x, rand_inds):
    q = _sc_gather_q(x, rand_inds)
    x_sq = jnp.sum(x * x, axis=1).reshape(1, N)
    i1, i2, i3 = _tc_topk(q, x, x_sq)
    return _sc_mean(x, i1, i2, i3)


# speedup vs baseline: 11.5168x; 1.0421x over previous
import functools

import jax
import jax.numpy as jnp
from jax import lax
from jax.experimental import pallas as pl
from jax.experimental.pallas import tpu as pltpu
from jax.experimental.pallas import tpu_sc as plsc

N = 4096
Q = 4096
D = 256
OUT_ROWS = 8192

NC, NS, L = 2, 16, 16
NW = NC * NS
BPW = Q // NW

_sc_mesh = plsc.VectorSubcoreMesh(core_axis_name="c", subcore_axis_name="s")


@functools.partial(
    pl.kernel,
    mesh=_sc_mesh,
    out_type=jax.ShapeDtypeStruct((Q, D), jnp.float32),
    scratch_types=[
        pltpu.VMEM((BPW,), jnp.int32),
        pltpu.VMEM((BPW, D), jnp.float32),
        pltpu.SemaphoreType.DMA,
    ],
)
def _sc_gather_q(x_hbm, inds_hbm, q_hbm, idx_v, rows_v, sem):
    wid = lax.axis_index("s") * NC + lax.axis_index("c")
    base = wid * BPW
    pltpu.sync_copy(inds_hbm.at[pl.ds(base, BPW)], idx_v)
    pltpu.async_copy(x_hbm.at[idx_v], rows_v, sem).wait()
    pltpu.sync_copy(rows_v, q_hbm.at[pl.ds(base, BPW)])


BQ = 512
GRID = Q // BQ


def _tc_topk_body(q_ref, x_ref, xsq_ref, i1_ref, i2_ref, i3_ref):
    qf = q_ref[...]
    q = qf.astype(jnp.bfloat16)
    x = x_ref[...].astype(jnp.bfloat16)
    mm = lax.dot_general(q, x, (((1,), (1,)), ((), ())),
                         preferred_element_type=jnp.float32)
    qsq = jnp.sum(qf * qf, axis=1, keepdims=True)
    s = (qsq - 2.0 * mm) + xsq_ref[...]
    cols = lax.broadcasted_iota(jnp.int32, (BQ, N), 1)
    for t, ref in enumerate((i1_ref, i2_ref, i3_ref)):
        m = jnp.min(s, axis=1, keepdims=True)
        idx = jnp.min(jnp.where(s == m, cols, N), axis=1, keepdims=True)
        ref[...] = idx[:, 0]
        if t < 2:
            s = jnp.where(cols == idx, jnp.float32(jnp.inf), s)


_tc_topk = pl.pallas_call(
    _tc_topk_body,
    grid=(GRID,),
    in_specs=[
        pl.BlockSpec((BQ, D), lambda i: (i, 0)),
        pl.BlockSpec((N, D), lambda i: (0, 0)),
        pl.BlockSpec((1, N), lambda i: (0, 0)),
    ],
    out_specs=[
        pl.BlockSpec((BQ,), lambda i: (i,)),
        pl.BlockSpec((BQ,), lambda i: (i,)),
        pl.BlockSpec((BQ,), lambda i: (i,)),
    ],
    out_shape=[jax.ShapeDtypeStruct((Q,), jnp.int32)] * 3,
)


@functools.partial(
    pl.kernel,
    mesh=_sc_mesh,
    out_type=jax.ShapeDtypeStruct((OUT_ROWS, D), jnp.float32),
    scratch_types=[
        pltpu.VMEM((BPW,), jnp.int32),
        pltpu.VMEM((BPW,), jnp.int32),
        pltpu.VMEM((BPW,), jnp.int32),
        pltpu.VMEM((BPW, D), jnp.float32),
        pltpu.VMEM((BPW, D), jnp.float32),
        pltpu.VMEM((BPW, D), jnp.float32),
        pltpu.SemaphoreType.DMA,
    ],
)
def _sc_mean(x_hbm, i1_hbm, i2_hbm, i3_hbm, out_hbm,
             i1_v, i2_v, i3_v, a_v, b_v, c_v, sem):
    wid = lax.axis_index("s") * NC + lax.axis_index("c")
    base = wid * BPW
    pltpu.sync_copy(x_hbm.at[pl.ds(base, BPW)], a_v)
    pltpu.sync_copy(a_v, out_hbm.at[pl.ds(base, BPW)])
    pltpu.sync_copy(i1_hbm.at[pl.ds(base, BPW)], i1_v)
    pltpu.sync_copy(i2_hbm.at[pl.ds(base, BPW)], i2_v)
    pltpu.sync_copy(i3_hbm.at[pl.ds(base, BPW)], i3_v)
    ca = pltpu.async_copy(x_hbm.at[i1_v], a_v, sem)
    cb = pltpu.async_copy(x_hbm.at[i2_v], b_v, sem)
    cc = pltpu.async_copy(x_hbm.at[i3_v], c_v, sem)
    ca.wait()
    cb.wait()
    cc.wait()
    third = jnp.float32(1.0 / 3.0)

    def row_body(r, carry):
        for ch in range(D // L):
            sl = pl.ds(ch * L, L)
            a_v[r, sl] = (a_v[r, sl] + b_v[r, sl] + c_v[r, sl]) * third
        return carry

    lax.fori_loop(0, BPW, row_body, 0)
    pltpu.sync_copy(a_v, out_hbm.at[pl.ds(N + base, BPW)])


def kernel(x, rand_inds):
    q = _sc_gather_q(x, rand_inds)
    x_sq = jnp.sum(x * x, axis=1).reshape(1, N)
    i1, i2, i3 = _tc_topk(q, x, x_sq)
    return _sc_mean(x, i1, i2, i3)
